# baseline (speedup 1.0000x reference)
import functools

import jax
import jax.numpy as jnp
from jax import lax
from jax.experimental import pallas as pl
from jax.experimental.pallas import tpu as pltpu
from jax.experimental.pallas import tpu_sc as plsc

B = 4096
N = 8192
NC = 2
NS = 16
NW = NC * NS
ROWS_PER_W = B // NW
K = 8
GROUPS = ROWS_PER_W // K
LANES = 16
CHUNK = 2048
CPG = N // CHUNK
STEPS = GROUPS * CPG
NV = CHUNK // LANES


def _masked_cumsum_body(x_hbm, m_hbm, out_hbm, xb, mb, ob, sx0, sm0, sx1, sm1):
    wid = lax.axis_index("s") * NC + lax.axis_index("c")
    base = wid * ROWS_PER_W
    sems = ((sx0, sm0), (sx1, sm1))

    def slices(s):
        row0 = base + (s // CPG) * K
        col0 = (s % CPG) * CHUNK
        return (pl.ds(row0, K), pl.ds(col0, CHUNK))

    for p in range(2):
        idx = slices(p)
        pltpu.make_async_copy(x_hbm.at[idx], xb.at[p], sems[p][0]).start()

    def pair(tt, carries):
        for p in range(2):
            s = 2 * tt + p
            idx = slices(s)
            pltpu.make_async_copy(x_hbm.at[idx], xb.at[p], sems[p][0]).wait()
            pltpu.sync_copy(m_hbm.at[idx], mb.at[p])

            fresh = (s % CPG) == 0
            carries = tuple(
                jnp.where(fresh, jnp.float32(0.0), c) for c in carries)

            def ibody(i, cs, p=p):
                col = pl.ds(i * LANES, LANES)
                new = []
                for k in range(K):
                    xm = xb[p, k, col] * mb[p, k, col]
                    sc = plsc.cumsum(xm)
                    ob[p, k, col] = sc + cs[k]
                    new.append(cs[k] + jnp.sum(xm))
                return tuple(new)

            carries = lax.fori_loop(0, NV, ibody, carries)

            @pl.when(s + 2 < STEPS)
            def _(p=p, s=s):
                nxt = slices(s + 2)
                pltpu.make_async_copy(
                    x_hbm.at[nxt], xb.at[p], sems[p][0]).start()

            pltpu.sync_copy(ob.at[p], out_hbm.at[idx])
        return carries

    lax.fori_loop(0, STEPS // 2, pair,
                  tuple(jnp.float32(0.0) for _ in range(K)))


_mesh = plsc.VectorSubcoreMesh(core_axis_name="c", subcore_axis_name="s")

_masked_cumsum = functools.partial(
    pl.kernel,
    out_type=jax.ShapeDtypeStruct((B, N), jnp.float32),
    mesh=_mesh,
    compiler_params=pltpu.CompilerParams(needs_layout_passes=False),
    scratch_types=[
        pltpu.VMEM((2, K, CHUNK), jnp.float32),
        pltpu.VMEM((2, K, CHUNK), jnp.float32),
        pltpu.VMEM((2, K, CHUNK), jnp.float32),
        pltpu.SemaphoreType.DMA,
        pltpu.SemaphoreType.DMA,
        pltpu.SemaphoreType.DMA,
        pltpu.SemaphoreType.DMA,
    ],
)(_masked_cumsum_body)


def kernel(x, mask):
    return _masked_cumsum(x, mask.astype(jnp.float32))


# speedup vs baseline: 2.4835x; 2.4835x over previous
import functools

import jax
import jax.numpy as jnp
from jax import lax
from jax.experimental import pallas as pl
from jax.experimental.pallas import tpu as pltpu
from jax.experimental.pallas import tpu_sc as plsc

B = 4096
N = 8192
NC = 2
NS = 16
NW = NC * NS
ROWS_PER_W = B // NW
K = 4
GROUPS = ROWS_PER_W // K
LANES = 16
HALF = N // 2
NV = HALF // LANES


def _masked_cumsum_body(x_hbm, m_hbm, out_hbm, xb, mb, ob,
                        sx0, sx1, sm0, sm1, so0, so1):
    wid = lax.axis_index("s") * NC + lax.axis_index("c")
    base = wid * ROWS_PER_W
    sx = (sx0, sx1)
    sm = (sm0, sm1)
    so = (so0, so1)

    def idx(g, h):
        return (pl.ds(base + g * K, K), pl.ds(h * HALF, HALF))

    def x_copy(g, h):
        return pltpu.make_async_copy(x_hbm.at[idx(g, h)], xb.at[h], sx[h])

    def m_copy(g, h):
        return pltpu.make_async_copy(m_hbm.at[idx(g, h)], mb.at[h], sm[h])

    def o_copy(g, h):
        return pltpu.make_async_copy(ob.at[h], out_hbm.at[idx(g, h)], so[h])

    x_copy(0, 0).start()
    m_copy(0, 0).start()

    def group(g, _):
        carries = tuple(jnp.float32(0.0) for _ in range(K))
        for h in range(2):
            x_copy(g, h).wait()
            m_copy(g, h).wait()
            if h == 0:
                x_copy(g, 1).start()
                m_copy(g, 1).start()
            else:
                @pl.when(g + 1 < GROUPS)
                def _():
                    x_copy(g + 1, 0).start()
                    m_copy(g + 1, 0).start()

            @pl.when(g > 0)
            def _(h=h):
                o_copy(g - 1, h).wait()

            def body(i, cs, h=h):
                col = pl.ds(i * LANES, LANES)
                new = []
                for k in range(K):
                    xm = xb[h, k, col] * mb[h, k, col]
                    s = plsc.cumsum(xm)
                    ob[h, k, col] = s + cs[k]
                    new.append(cs[k] + jnp.sum(xm))
                return tuple(new)

            carries = lax.fori_loop(0, NV, body, carries)
            o_copy(g, h).start()
        return 0

    lax.fori_loop(0, GROUPS, group, 0)
    for h in range(2):
        o_copy(GROUPS - 1, h).wait()


_mesh = plsc.VectorSubcoreMesh(core_axis_name="c", subcore_axis_name="s")

_masked_cumsum = functools.partial(
    pl.kernel,
    out_type=jax.ShapeDtypeStruct((B, N), jnp.float32),
    mesh=_mesh,
    compiler_params=pltpu.CompilerParams(needs_layout_passes=False),
    scratch_types=[
        pltpu.VMEM((2, K, HALF), jnp.float32),
        pltpu.VMEM((2, K, HALF), jnp.float32),
        pltpu.VMEM((2, K, HALF), jnp.float32),
        pltpu.SemaphoreType.DMA,
        pltpu.SemaphoreType.DMA,
        pltpu.SemaphoreType.DMA,
        pltpu.SemaphoreType.DMA,
        pltpu.SemaphoreType.DMA,
        pltpu.SemaphoreType.DMA,
    ],
)(_masked_cumsum_body)


def kernel(x, mask):
    return _masked_cumsum(x, mask.astype(jnp.float32))
